# baseline (speedup 1.0000x reference)
import jax
import jax.numpy as jnp
from jax.experimental import pallas as pl
from jax.experimental.pallas import tpu as pltpu

_BLOCK_COLS = 2048
_K_PLANES = 8


def _tc_run(kT, loc, scale, eT, n_rows, n_cols):
    grid = (n_cols // _BLOCK_COLS,)

    def body(loc_ref, scl_ref, k_ref, e_ref, o_ref):
        kv = k_ref[...]
        ev = e_ref[...]
        acc = jnp.zeros_like(ev)
        for i in range(_K_PLANES):
            acc = jnp.where(kv == i, loc_ref[i] + scl_ref[i] * ev, acc)
        o_ref[...] = acc

    block = pl.BlockSpec((n_rows, _BLOCK_COLS), lambda i: (0, i))
    return pl.pallas_call(
        body,
        grid=grid,
        in_specs=[
            pl.BlockSpec(memory_space=pltpu.SMEM),
            pl.BlockSpec(memory_space=pltpu.SMEM),
            block,
            block,
        ],
        out_specs=block,
        out_shape=jax.ShapeDtypeStruct((n_rows, n_cols), jnp.float32),
    )(loc, scale, kT, eT)


def kernel(k, loc, scale, eps):
    n_rows, n_cols = k.shape
    outT = _tc_run(k.astype(jnp.int32).T, loc.astype(jnp.float32),
                   scale.astype(jnp.float32), eps.astype(jnp.float32).T,
                   n_cols, n_rows)
    return outT.T


# speedup vs baseline: 1.0391x; 1.0391x over previous
import jax
import jax.numpy as jnp
from jax.experimental import pallas as pl
from jax.experimental.pallas import tpu as pltpu

_BLOCK_COLS = 4096
_K_PLANES = 8


def _tc_run(kT, loc, scale, eT, n_rows, n_cols):
    grid = (n_cols // _BLOCK_COLS,)

    def body(loc_ref, scl_ref, k_ref, e_ref, o_ref):
        kv = k_ref[...]
        ev = e_ref[...]
        acc = jnp.zeros_like(ev)
        for i in range(_K_PLANES):
            acc = jnp.where(kv == i, loc_ref[i] + scl_ref[i] * ev, acc)
        o_ref[...] = acc

    block = pl.BlockSpec((n_rows, _BLOCK_COLS), lambda i: (0, i))
    return pl.pallas_call(
        body,
        grid=grid,
        in_specs=[
            pl.BlockSpec(memory_space=pltpu.SMEM),
            pl.BlockSpec(memory_space=pltpu.SMEM),
            block,
            block,
        ],
        out_specs=block,
        out_shape=jax.ShapeDtypeStruct((n_rows, n_cols), jnp.float32),
    )(loc, scale, kT, eT)


def kernel(k, loc, scale, eps):
    n_rows, n_cols = k.shape
    outT = _tc_run(k.astype(jnp.int32).T, loc.astype(jnp.float32),
                   scale.astype(jnp.float32), eps.astype(jnp.float32).T,
                   n_cols, n_rows)
    return outT.T
